# baseline (speedup 1.0000x reference)
import functools

import jax
import jax.numpy as jnp
from jax import lax
from jax.experimental import pallas as pl
from jax.experimental.pallas import tpu as pltpu
from jax.experimental.pallas import tpu_sc as plsc

BATCH = 256
MAXLEN = 50
CLIP = 30
COPYLEN = 32
TCHUNK = 8
NCHUNK = COPYLEN // TCHUNK
DIM = 128
LANES = 16
NVEC = DIM // LANES


def _recip_vec(den_f32_scalar):
    nf = jnp.broadcast_to(den_f32_scalar, (LANES,))
    seed = jnp.asarray(0x7EF311C3, jnp.int32) - lax.bitcast_convert_type(
        nf, jnp.int32
    )
    y = lax.bitcast_convert_type(seed, jnp.float32)
    two = jnp.full((LANES,), 2.0, jnp.float32)
    y = y * (two - nf * y)
    y = y * (two - nf * y)
    y = y * (two - nf * y)
    return y


def _make_kernel():
    info = plsc.get_sparse_core_info()
    nc, ns = info.num_cores, info.num_subcores
    nw = nc * ns
    rows_per_w = BATCH // nw

    mesh = plsc.VectorSubcoreMesh(core_axis_name="c", subcore_axis_name="s")

    @functools.partial(
        pl.kernel,
        mesh=mesh,
        out_type=jax.ShapeDtypeStruct((BATCH, DIM), jnp.float32),
        scratch_types=[
            pltpu.VMEM((BATCH + LANES,), jnp.int32),
            pltpu.VMEM((COPYLEN, 8, DIM), jnp.float32),
            pltpu.VMEM((rows_per_w, DIM), jnp.float32),
            pltpu.SemaphoreType.DMA,
        ],
    )
    def seq_mean(xt_hbm, len_hbm, out_hbm, len_v, buf_v, out_v, sem):
        wid = lax.axis_index("s") * nc + lax.axis_index("c")
        base = wid * rows_per_w

        pltpu.sync_copy(len_hbm, len_v.at[pl.ds(0, BATCH)])
        copies = [
            pltpu.async_copy(
                xt_hbm.at[pl.ds(c * TCHUNK, TCHUNK), pl.ds(base, rows_per_w)],
                buf_v.at[pl.ds(c * TCHUNK, TCHUNK)],
                sem,
            )
            for c in range(NCHUNK)
        ]

        zeros = tuple(jnp.zeros((LANES,), jnp.float32) for _ in range(NVEC))

        for c in range(NCHUNK):
            copies[c].wait()
            lo = c * TCHUNK

            def row_body(r, _, _lo=lo, _last=(c == NCHUNK - 1)):
                ln = len_v[pl.ds(base + r, LANES)][0]
                lnc = jnp.minimum(ln, CLIP)
                if _lo == 0:
                    accs = zeros
                else:
                    accs = tuple(
                        out_v[r, pl.ds(k * LANES, LANES)] for k in range(NVEC)
                    )

                def t_body(i, a, _lo=_lo):
                    t0 = _lo + 2 * i
                    for t in (t0, t0 + 1):
                        keep = t < lnc
                        a = tuple(
                            ak
                            + jnp.where(
                                keep,
                                buf_v[t, r, pl.ds(k * LANES, LANES)],
                                zeros[0],
                            )
                            for k, ak in enumerate(a)
                        )
                    return a

                accs = lax.fori_loop(0, TCHUNK // 2, t_body, accs)
                if _last:
                    den = jnp.maximum(lnc, 1).astype(jnp.float32)
                    scale = _recip_vec(den)
                    accs = tuple(ak * scale for ak in accs)
                for k in range(NVEC):
                    out_v[r, pl.ds(k * LANES, LANES)] = accs[k]
                return 0

            lax.fori_loop(0, rows_per_w, row_body, 0)

        pltpu.sync_copy(out_v, out_hbm.at[pl.ds(base, rows_per_w)])

    return seq_mean


_seq_mean = _make_kernel()


def kernel(opt_seq_embedding, length):
    xt = jnp.transpose(opt_seq_embedding, (1, 0, 2))
    return _seq_mean(xt, length)


# speedup vs baseline: 1.0158x; 1.0158x over previous
import functools

import jax
import jax.numpy as jnp
from jax import lax
from jax.experimental import pallas as pl
from jax.experimental.pallas import tpu as pltpu
from jax.experimental.pallas import tpu_sc as plsc

BATCH = 256
MAXLEN = 50
CLIP = 30
COPYLEN = 30
TCHUNK = 15
NCHUNK = COPYLEN // TCHUNK
DIM = 128
LANES = 16
NVEC = DIM // LANES


def _recip_vec(den_f32_scalar):
    nf = jnp.broadcast_to(den_f32_scalar, (LANES,))
    seed = jnp.asarray(0x7EF311C3, jnp.int32) - lax.bitcast_convert_type(
        nf, jnp.int32
    )
    y = lax.bitcast_convert_type(seed, jnp.float32)
    two = jnp.full((LANES,), 2.0, jnp.float32)
    y = y * (two - nf * y)
    y = y * (two - nf * y)
    y = y * (two - nf * y)
    return y


def _make_kernel():
    info = plsc.get_sparse_core_info()
    nc, ns = info.num_cores, info.num_subcores
    nw = nc * ns
    rows_per_w = BATCH // nw

    mesh = plsc.VectorSubcoreMesh(core_axis_name="c", subcore_axis_name="s")

    @functools.partial(
        pl.kernel,
        mesh=mesh,
        out_type=jax.ShapeDtypeStruct((BATCH, DIM), jnp.float32),
        scratch_types=[
            pltpu.VMEM((BATCH + LANES,), jnp.int32),
            pltpu.VMEM((COPYLEN, 8, DIM), jnp.float32),
            pltpu.VMEM((rows_per_w, DIM), jnp.float32),
            pltpu.SemaphoreType.DMA,
        ],
    )
    def seq_mean(xt_hbm, len_hbm, out_hbm, len_v, buf_v, out_v, sem):
        wid = lax.axis_index("s") * nc + lax.axis_index("c")
        base = wid * rows_per_w

        pltpu.sync_copy(len_hbm, len_v.at[pl.ds(0, BATCH)])
        copies = [
            pltpu.async_copy(
                xt_hbm.at[pl.ds(c * TCHUNK, TCHUNK), pl.ds(base, rows_per_w)],
                buf_v.at[pl.ds(c * TCHUNK, TCHUNK)],
                sem,
            )
            for c in range(NCHUNK)
        ]

        zeros = tuple(jnp.zeros((LANES,), jnp.float32) for _ in range(NVEC))

        for c in range(NCHUNK):
            copies[c].wait()
            lo = c * TCHUNK

            def row_body(r, _, _lo=lo, _last=(c == NCHUNK - 1)):
                ln = len_v[pl.ds(base + r, LANES)][0]
                lnc = jnp.minimum(ln, CLIP)
                hi = jnp.maximum(jnp.minimum(lnc, _lo + TCHUNK), _lo)
                if _lo == 0:
                    accs = zeros
                else:
                    accs = tuple(
                        out_v[r, pl.ds(k * LANES, LANES)] for k in range(NVEC)
                    )

                def t_body(t, a):
                    return tuple(
                        ak + buf_v[t, r, pl.ds(k * LANES, LANES)]
                        for k, ak in enumerate(a)
                    )

                accs = lax.fori_loop(_lo, hi, t_body, accs)
                if _last:
                    den = jnp.maximum(lnc, 1).astype(jnp.float32)
                    scale = _recip_vec(den)
                    accs = tuple(ak * scale for ak in accs)
                for k in range(NVEC):
                    out_v[r, pl.ds(k * LANES, LANES)] = accs[k]
                return 0

            lax.fori_loop(0, rows_per_w, row_body, 0)

        pltpu.sync_copy(out_v, out_hbm.at[pl.ds(base, rows_per_w)])

    return seq_mean


_seq_mean = _make_kernel()


def kernel(opt_seq_embedding, length):
    xt = jnp.transpose(opt_seq_embedding, (1, 0, 2))
    return _seq_mean(xt, length)
